# baseline (speedup 1.0000x reference)
import jax
import jax.numpy as jnp
from jax import lax
from jax.experimental import pallas as pl
from jax.experimental.pallas import tpu as pltpu
from jax.experimental.pallas import tpu_sc as plsc

NUM_E = 1000000
D = 32
BATCH = 16384
HIST = 50
NC = 2
NS = 16
NW = NC * NS
PER_W = BATCH // NW
NBUF = 4


def _body(table_hbm, idx_hbm, out_hbm, idx_v, rows_v, gsem, wsem):
  wid = lax.axis_index("s") * NC + lax.axis_index("c")
  base = wid * PER_W
  pltpu.sync_copy(idx_hbm.at[pl.ds(base, PER_W)], idx_v)

  def gather_start(e, b):
    pltpu.async_copy(table_hbm.at[idx_v.at[e]], rows_v.at[b], gsem.at[b])

  def gather_wait(e, b):
    pltpu.make_async_copy(
        table_hbm.at[idx_v.at[e]], rows_v.at[b], gsem.at[b]).wait()

  def write_start(e, b):
    pltpu.async_copy(rows_v.at[b], out_hbm.at[base + e], wsem.at[b])

  def write_wait(e, b):
    pltpu.make_async_copy(
        rows_v.at[b], out_hbm.at[base + e], wsem.at[b]).wait()

  for b in range(NBUF):
    gather_start(b, b)

  def outer(e0, _):
    for b in range(NBUF):
      e = e0 * NBUF + b
      gather_wait(e, b)
      write_start(e, b)
      write_wait(e, b)

      @pl.when(e + NBUF < PER_W)
      def _():
        gather_start(e + NBUF, b)

    return 0

  lax.fori_loop(0, PER_W // NBUF, outer, 0)


def kernel(input, embedding_weight):
  mesh = plsc.VectorSubcoreMesh(core_axis_name="c", subcore_axis_name="s")
  table16 = embedding_weight.astype(jnp.bfloat16)
  return pl.kernel(
      _body,
      out_type=jax.ShapeDtypeStruct((BATCH, HIST, D), jnp.bfloat16),
      mesh=mesh,
      compiler_params=pltpu.CompilerParams(
          needs_layout_passes=False, use_tc_tiling_on_sc=False),
      scratch_types=[
          pltpu.VMEM((PER_W, HIST), jnp.int32),
          pltpu.VMEM((NBUF, HIST, D), jnp.bfloat16),
          pltpu.SemaphoreType.DMA((NBUF,)),
          pltpu.SemaphoreType.DMA((NBUF,)),
      ],
  )(table16, input.astype(jnp.int32))


# speedup vs baseline: 1.1277x; 1.1277x over previous
import jax
import jax.numpy as jnp
from jax import lax
from jax.experimental import pallas as pl
from jax.experimental.pallas import tpu as pltpu
from jax.experimental.pallas import tpu_sc as plsc

NUM_E = 1000000
D = 32
BATCH = 16384
HIST = 50
NC = 2
NS = 16
NW = NC * NS
PER_W = BATCH // NW
NBUF = 8


def _rtne16(u):
  odd = lax.shift_right_logical(u, 16) & 1
  return lax.shift_right_logical(u + 0x7FFF + odd, 16)


def _body(table_hbm, idx_hbm, out_hbm, idx_v, rows_v, obuf_v, gsem, wsem):
  wid = lax.axis_index("s") * NC + lax.axis_index("c")
  base = wid * PER_W
  pltpu.sync_copy(idx_hbm.at[pl.ds(base, PER_W)], idx_v)

  col_even = 2 * lax.iota(jnp.int32, 16)
  col_odd = col_even + 1

  def gather_start(e, b):
    pltpu.async_copy(table_hbm.at[idx_v.at[e]], rows_v.at[b], gsem.at[b])

  def gather_wait(e, b):
    pltpu.make_async_copy(
        table_hbm.at[idx_v.at[e]], rows_v.at[b], gsem.at[b]).wait()

  def write_start(e, b):
    pltpu.async_copy(obuf_v.at[b], out_hbm.at[base + e], wsem.at[b])

  def write_wait(e, b):
    pltpu.make_async_copy(
        obuf_v.at[b], out_hbm.at[base + e], wsem.at[b]).wait()

  for b in range(NBUF):
    gather_start(b, b)

  def outer(e0, _):
    for b in range(NBUF):
      e = e0 * NBUF + b
      gather_wait(e, b)

      @pl.when(e >= NBUF)
      def _():
        write_wait(e - NBUF, b)

      rb = rows_v.at[b]
      ob = obuf_v.at[b]

      @plsc.parallel_loop(0, HIST, unroll=2)
      def _row(r):
        rv = rb.at[r]
        a = plsc.bitcast(plsc.load_gather(rv, [col_even]), jnp.int32)
        c = plsc.bitcast(plsc.load_gather(rv, [col_odd]), jnp.int32)
        word = _rtne16(a) | lax.shift_left(_rtne16(c), 16)
        ob.at[r][:] = plsc.bitcast(word, jnp.bfloat16)

      write_start(e, b)

      @pl.when(e + NBUF < PER_W)
      def _():
        gather_start(e + NBUF, b)

    return 0

  lax.fori_loop(0, PER_W // NBUF, outer, 0)
  for b in range(NBUF):
    write_wait(PER_W - NBUF + b, b)


def kernel(input, embedding_weight):
  mesh = plsc.VectorSubcoreMesh(core_axis_name="c", subcore_axis_name="s")
  return pl.kernel(
      _body,
      out_type=jax.ShapeDtypeStruct((BATCH, HIST, D), jnp.bfloat16),
      mesh=mesh,
      compiler_params=pltpu.CompilerParams(
          needs_layout_passes=False, use_tc_tiling_on_sc=False),
      scratch_types=[
          pltpu.VMEM((PER_W, HIST), jnp.int32),
          pltpu.VMEM((NBUF, HIST, D), jnp.float32),
          pltpu.VMEM((NBUF, HIST, D), jnp.bfloat16),
          pltpu.SemaphoreType.DMA((NBUF,)),
          pltpu.SemaphoreType.DMA((NBUF,)),
      ],
  )(embedding_weight, input.astype(jnp.int32))
